# Initial kernel scaffold; baseline (speedup 1.0000x reference)
import jax
import jax.numpy as jnp
from jax import lax
from jax.experimental import pallas as pl
from jax.experimental.pallas import tpu as pltpu
from jax.experimental.pallas import tpu_sc as plsc

_N = 10000
_E = 320000
_D = 64
_NC, _NS = 2, 16
_NW = _NC * _NS
_EPT = _E // _NW
_CH = 80
_NCH = _EPT // _CH
_RPT = _N // _NS
_ZR = 125
_RB = 1000

_mesh = plsc.VectorSubcoreMesh(
    core_axis_name="c", subcore_axis_name="s",
    num_cores=_NC, num_subcores=_NS)


def _deg_body(dst_hbm, out_hbm, dstv, onesv, zbuf, acc):
    c = lax.axis_index("c")
    s = lax.axis_index("s")
    wid = s * _NC + c

    @pl.loop(0, _CH)
    def _fill(i):
        onesv[i, :] = jnp.full((16,), 1.0, dtype=jnp.float32)

    @pl.loop(0, _RPT)
    def _zero(i):
        zbuf[i, :] = jnp.zeros((16,), dtype=jnp.float32)

    pltpu.sync_copy(dst_hbm.at[wid], dstv)
    pltpu.sync_copy(zbuf, acc.at[pl.ds(s * _RPT, _RPT)])
    plsc.subcore_barrier()

    @pl.loop(0, _NCH)
    def _scat(i):
        pltpu.sync_copy(onesv, acc.at[dstv.at[i]], add=True)

    plsc.subcore_barrier()
    pltpu.sync_copy(acc.at[pl.ds(s * _RPT, _RPT)],
                    out_hbm.at[c].at[pl.ds(s * _RPT, _RPT)])


def _deg_partials(dst):
    return pl.kernel(
        _deg_body,
        out_type=jax.ShapeDtypeStruct((_NC, _N, 16), jnp.float32),
        mesh=_mesh,
        scratch_types=[
            pltpu.VMEM((_NCH, _CH), jnp.int32),
            pltpu.VMEM((_CH, 16), jnp.float32),
            pltpu.VMEM((_RPT, 16), jnp.float32),
            pltpu.VMEM_SHARED((_N, 16), jnp.float32),
        ],
    )(dst)


def _agg_body(g_hbm, src_hbm, dst_hbm, out_hbm,
              srcv, dstv, buf0, buf1, zbuf, acc, sem0, sem1):
    c = lax.axis_index("c")
    s = lax.axis_index("s")
    wid = s * _NC + c

    @pl.loop(0, _ZR)
    def _zero(i):
        @pl.loop(0, _D // 16)
        def _zero_in(j):
            zbuf[i, pl.ds(j * 16, 16)] = jnp.zeros((16,), dtype=jnp.float32)

    @pl.loop(0, _RPT // _ZR)
    def _zacc(j):
        pltpu.sync_copy(zbuf, acc.at[pl.ds(s * _RPT + j * _ZR, _ZR)])

    pltpu.sync_copy(src_hbm.at[wid], srcv)
    pltpu.sync_copy(dst_hbm.at[wid], dstv)
    plsc.subcore_barrier()

    pltpu.async_copy(g_hbm.at[srcv.at[0]], buf0, sem0)

    @pl.loop(0, _NCH, step=2)
    def _pipe(i):
        @pl.when(i + 1 < _NCH)
        def _():
            pltpu.async_copy(g_hbm.at[srcv.at[i + 1]], buf1, sem1)

        pltpu.make_async_copy(g_hbm.at[srcv.at[i]], buf0, sem0).wait()
        pltpu.sync_copy(buf0, acc.at[dstv.at[i]], add=True)

        @pl.when(i + 2 < _NCH)
        def _():
            pltpu.async_copy(g_hbm.at[srcv.at[i + 2]], buf0, sem0)

        @pl.when(i + 1 < _NCH)
        def _():
            pltpu.make_async_copy(g_hbm.at[srcv.at[i + 1]], buf1, sem1).wait()
            pltpu.sync_copy(buf1, acc.at[dstv.at[i + 1]], add=True)

    plsc.subcore_barrier()
    pltpu.sync_copy(acc.at[pl.ds(s * _RPT, _RPT)],
                    out_hbm.at[c].at[pl.ds(s * _RPT, _RPT)])


def _aggregate(g, src, dst):
    return pl.kernel(
        _agg_body,
        out_type=jax.ShapeDtypeStruct((_NC, _N, _D), jnp.float32),
        mesh=_mesh,
        scratch_types=[
            pltpu.VMEM((_NCH, _CH), jnp.int32),
            pltpu.VMEM((_NCH, _CH), jnp.int32),
            pltpu.VMEM((_CH, _D), jnp.float32),
            pltpu.VMEM((_CH, _D), jnp.float32),
            pltpu.VMEM((_ZR, _D), jnp.float32),
            pltpu.VMEM_SHARED((_N, _D), jnp.float32),
            pltpu.SemaphoreType.DMA,
            pltpu.SemaphoreType.DMA,
        ],
    )(g, src, dst)


def _matmul1(x, W1):
    def body(x_ref, w_ref, o_ref):
        o_ref[...] = jnp.dot(x_ref[...], w_ref[...],
                             preferred_element_type=jnp.float32)

    return pl.pallas_call(
        body,
        grid=(_N // _RB,),
        in_specs=[pl.BlockSpec((_RB, 128), lambda i: (i, 0)),
                  pl.BlockSpec((128, _D), lambda i: (0, 0))],
        out_specs=pl.BlockSpec((_RB, _D), lambda i: (i, 0)),
        out_shape=jax.ShapeDtypeStruct((_N, _D), jnp.float32),
    )(x, W1)


def _combine1(degp, h1, b1):
    def body(p_ref, h_ref, b_ref, g_ref, st_ref, dis_ref):
        deg = 1.0 + p_ref[0] + p_ref[1]
        dcol = lax.rsqrt(deg[:, 0:1])
        disb = jnp.broadcast_to(dcol, (_RB, _D))
        h = h_ref[...]
        g_ref[...] = h * disb
        st_ref[...] = h * (dcol * dcol) + b_ref[...]
        dis_ref[...] = disb

    return pl.pallas_call(
        body,
        grid=(_N // _RB,),
        in_specs=[pl.BlockSpec((_NC, _RB, 16), lambda i: (0, i, 0)),
                  pl.BlockSpec((_RB, _D), lambda i: (i, 0)),
                  pl.BlockSpec((1, _D), lambda i: (0, 0))],
        out_specs=[pl.BlockSpec((_RB, _D), lambda i: (i, 0))] * 3,
        out_shape=[jax.ShapeDtypeStruct((_N, _D), jnp.float32)] * 3,
    )(degp, h1, b1)


def _combine2(s1, disb, st1):
    def body(p_ref, d_ref, st_ref, g_ref, st2_ref):
        dis = d_ref[...]
        h = jnp.maximum(dis * (p_ref[0] + p_ref[1]) + st_ref[...], 0.0)
        g_ref[...] = h * dis
        st2_ref[...] = h * dis * dis

    return pl.pallas_call(
        body,
        grid=(_N // _RB,),
        in_specs=[pl.BlockSpec((_NC, _RB, _D), lambda i: (0, i, 0)),
                  pl.BlockSpec((_RB, _D), lambda i: (i, 0)),
                  pl.BlockSpec((_RB, _D), lambda i: (i, 0))],
        out_specs=[pl.BlockSpec((_RB, _D), lambda i: (i, 0))] * 2,
        out_shape=[jax.ShapeDtypeStruct((_N, _D), jnp.float32)] * 2,
    )(s1, disb, st1)


def _final(s2, disb, st2, W2, b2):
    def body(p_ref, d_ref, st_ref, w_ref, b_ref, o_ref):
        a2 = d_ref[...] * (p_ref[0] + p_ref[1]) + st_ref[...]
        o_ref[...] = jnp.dot(a2, w_ref[...],
                             preferred_element_type=jnp.float32) + b_ref[...]

    return pl.pallas_call(
        body,
        grid=(_N // _RB,),
        in_specs=[pl.BlockSpec((_NC, _RB, _D), lambda i: (0, i, 0)),
                  pl.BlockSpec((_RB, _D), lambda i: (i, 0)),
                  pl.BlockSpec((_RB, _D), lambda i: (i, 0)),
                  pl.BlockSpec((_D, 128), lambda i: (0, 0)),
                  pl.BlockSpec((1, 128), lambda i: (0, 0))],
        out_specs=pl.BlockSpec((_RB, 128), lambda i: (i, 0)),
        out_shape=jax.ShapeDtypeStruct((_N, 128), jnp.float32),
    )(s2, disb, st2, W2, b2)


def kernel(x, edge_index, W1, b1, W2, b2):
    src = edge_index[0].astype(jnp.int32).reshape(_NW, _NCH, _CH)
    dst = edge_index[1].astype(jnp.int32).reshape(_NW, _NCH, _CH)

    degp = _deg_partials(dst)
    h1 = _matmul1(x, W1)

    g1, st1, disb = _combine1(degp, h1, b1.reshape(1, _D))
    s1 = _aggregate(g1, src, dst)
    g2, st2 = _combine2(s1, disb, st1)
    s2 = _aggregate(g2, src, dst)
    return _final(s2, disb, st2, W2, b2.reshape(1, 128))


# speedup vs baseline: 31.4232x; 31.4232x over previous
import jax
import jax.numpy as jnp
from jax import lax
from jax.experimental import pallas as pl
from jax.experimental.pallas import tpu as pltpu
from jax.experimental.pallas import tpu_sc as plsc

_N = 10000
_E = 320000
_D = 64
_NC, _NS = 2, 16
_NW = _NC * _NS
_EPT = _E // _NW
_CH = 80
_NCH = _EPT // _CH
_NP = 10240
_RPT = _NP // _NS
_ZR = 128
_RB = 1000

_mesh = plsc.VectorSubcoreMesh(
    core_axis_name="c", subcore_axis_name="s",
    num_cores=_NC, num_subcores=_NS)


def _deg_body(dst_hbm, out_hbm, dstv, onesv, zbuf, acc):
    c = lax.axis_index("c")
    s = lax.axis_index("s")
    wid = s * _NC + c

    @pl.loop(0, _CH)
    def _fill(i):
        onesv[i, :] = jnp.full((16,), 1.0, dtype=jnp.float32)

    @pl.loop(0, _RPT)
    def _zero(i):
        zbuf[i, :] = jnp.zeros((16,), dtype=jnp.float32)

    pltpu.sync_copy(dst_hbm.at[wid], dstv)
    pltpu.sync_copy(zbuf, acc.at[pl.ds(s * _RPT, _RPT)])
    plsc.subcore_barrier()

    @pl.loop(0, _NCH)
    def _scat(i):
        pltpu.sync_copy(onesv, acc.at[dstv.at[i]], add=True)

    plsc.subcore_barrier()
    pltpu.sync_copy(acc.at[pl.ds(s * _RPT, _RPT)],
                    out_hbm.at[c].at[pl.ds(s * _RPT, _RPT)])


def _deg_partials(dst):
    return pl.kernel(
        _deg_body,
        out_type=jax.ShapeDtypeStruct((_NC, _NP, 16), jnp.float32),
        mesh=_mesh,
        compiler_params=pltpu.CompilerParams(use_tc_tiling_on_sc=False),
        scratch_types=[
            pltpu.VMEM((_NCH, _CH), jnp.int32),
            pltpu.VMEM((_CH, 16), jnp.float32),
            pltpu.VMEM((_RPT, 16), jnp.float32),
            pltpu.VMEM_SHARED((_NP, 16), jnp.float32),
        ],
    )(dst)


def _agg_body(g_hbm, src_hbm, dst_hbm, out_hbm,
              srcv, dstv, buf0, buf1, zbuf, acc, sem0, sem1):
    c = lax.axis_index("c")
    s = lax.axis_index("s")
    wid = s * _NC + c

    @pl.loop(0, _ZR)
    def _zero(i):
        @pl.loop(0, _D // 16)
        def _zero_in(j):
            zbuf[i, pl.ds(j * 16, 16)] = jnp.zeros((16,), dtype=jnp.float32)

    @pl.loop(0, _RPT // _ZR)
    def _zacc(j):
        pltpu.sync_copy(zbuf, acc.at[pl.ds(s * _RPT + j * _ZR, _ZR)])

    pltpu.sync_copy(src_hbm.at[wid], srcv)
    pltpu.sync_copy(dst_hbm.at[wid], dstv)
    plsc.subcore_barrier()

    @pl.loop(0, _NCH - 1, step=2)
    def _pipe(i):
        d0 = pltpu.async_copy(g_hbm.at[srcv.at[i]], buf0, sem0)
        d1 = pltpu.async_copy(g_hbm.at[srcv.at[i + 1]], buf1, sem1)
        d0.wait()
        pltpu.sync_copy(buf0, acc.at[dstv.at[i]], add=True)
        d1.wait()
        pltpu.sync_copy(buf1, acc.at[dstv.at[i + 1]], add=True)

    pltpu.async_copy(g_hbm.at[srcv.at[_NCH - 1]], buf0, sem0).wait()
    pltpu.sync_copy(buf0, acc.at[dstv.at[_NCH - 1]], add=True)

    plsc.subcore_barrier()
    pltpu.sync_copy(acc.at[pl.ds(s * _RPT, _RPT)],
                    out_hbm.at[c].at[pl.ds(s * _RPT, _RPT)])


def _aggregate(g, src, dst):
    return pl.kernel(
        _agg_body,
        out_type=jax.ShapeDtypeStruct((_NC, _NP, _D), jnp.float32),
        mesh=_mesh,
        compiler_params=pltpu.CompilerParams(use_tc_tiling_on_sc=False),
        scratch_types=[
            pltpu.VMEM((_NCH, _CH), jnp.int32),
            pltpu.VMEM((_NCH, _CH), jnp.int32),
            pltpu.VMEM((_CH, _D), jnp.float32),
            pltpu.VMEM((_CH, _D), jnp.float32),
            pltpu.VMEM((_ZR, _D), jnp.float32),
            pltpu.VMEM_SHARED((_NP, _D), jnp.float32),
            pltpu.SemaphoreType.DMA,
            pltpu.SemaphoreType.DMA,
        ],
    )(g, src, dst)


def _matmul1(x, W1):
    def body(x_ref, w_ref, o_ref):
        o_ref[...] = jnp.dot(x_ref[...], w_ref[...],
                             preferred_element_type=jnp.float32)

    return pl.pallas_call(
        body,
        grid=(_N // _RB,),
        in_specs=[pl.BlockSpec((_RB, 128), lambda i: (i, 0)),
                  pl.BlockSpec((128, _D), lambda i: (0, 0))],
        out_specs=pl.BlockSpec((_RB, _D), lambda i: (i, 0)),
        out_shape=jax.ShapeDtypeStruct((_N, _D), jnp.float32),
    )(x, W1)


def _combine1(degp, h1, b1):
    def body(p_ref, h_ref, b_ref, g_ref, st_ref, dis_ref):
        deg = 1.0 + p_ref[0] + p_ref[1]
        dcol = lax.rsqrt(deg[:, 0:1])
        disb = jnp.broadcast_to(dcol, (_RB, _D))
        h = h_ref[...]
        g_ref[...] = h * disb
        st_ref[...] = h * (dcol * dcol) + b_ref[...]
        dis_ref[...] = disb

    return pl.pallas_call(
        body,
        grid=(_N // _RB,),
        in_specs=[pl.BlockSpec((_NC, _RB, 16), lambda i: (0, i, 0)),
                  pl.BlockSpec((_RB, _D), lambda i: (i, 0)),
                  pl.BlockSpec((1, _D), lambda i: (0, 0))],
        out_specs=[pl.BlockSpec((_RB, _D), lambda i: (i, 0))] * 3,
        out_shape=[jax.ShapeDtypeStruct((_N, _D), jnp.float32)] * 3,
    )(degp, h1, b1)


def _combine2(s1, disb, st1):
    def body(p_ref, d_ref, st_ref, g_ref, st2_ref):
        dis = d_ref[...]
        h = jnp.maximum(dis * (p_ref[0] + p_ref[1]) + st_ref[...], 0.0)
        g_ref[...] = h * dis
        st2_ref[...] = h * dis * dis

    return pl.pallas_call(
        body,
        grid=(_N // _RB,),
        in_specs=[pl.BlockSpec((_NC, _RB, _D), lambda i: (0, i, 0)),
                  pl.BlockSpec((_RB, _D), lambda i: (i, 0)),
                  pl.BlockSpec((_RB, _D), lambda i: (i, 0))],
        out_specs=[pl.BlockSpec((_RB, _D), lambda i: (i, 0))] * 2,
        out_shape=[jax.ShapeDtypeStruct((_N, _D), jnp.float32)] * 2,
    )(s1, disb, st1)


def _final(s2, disb, st2, W2, b2):
    def body(p_ref, d_ref, st_ref, w_ref, b_ref, o_ref):
        a2 = d_ref[...] * (p_ref[0] + p_ref[1]) + st_ref[...]
        o_ref[...] = jnp.dot(a2, w_ref[...],
                             preferred_element_type=jnp.float32) + b_ref[...]

    return pl.pallas_call(
        body,
        grid=(_N // _RB,),
        in_specs=[pl.BlockSpec((_NC, _RB, _D), lambda i: (0, i, 0)),
                  pl.BlockSpec((_RB, _D), lambda i: (i, 0)),
                  pl.BlockSpec((_RB, _D), lambda i: (i, 0)),
                  pl.BlockSpec((_D, 128), lambda i: (0, 0)),
                  pl.BlockSpec((1, 128), lambda i: (0, 0))],
        out_specs=pl.BlockSpec((_RB, 128), lambda i: (i, 0)),
        out_shape=jax.ShapeDtypeStruct((_N, 128), jnp.float32),
    )(s2, disb, st2, W2, b2)


def kernel(x, edge_index, W1, b1, W2, b2):
    src = edge_index[0].astype(jnp.int32).reshape(_NW, _NCH, _CH)
    dst = edge_index[1].astype(jnp.int32).reshape(_NW, _NCH, _CH)

    degp = _deg_partials(dst)
    h1 = _matmul1(x, W1)

    g1, st1, disb = _combine1(degp, h1, b1.reshape(1, _D))
    s1 = _aggregate(g1, src, dst)
    g2, st2 = _combine2(s1, disb, st1)
    s2 = _aggregate(g2, src, dst)
    return _final(s2, disb, st2, W2, b2.reshape(1, 128))


# speedup vs baseline: 37.0350x; 1.1786x over previous
import jax
import jax.numpy as jnp
from jax import lax
from jax.experimental import pallas as pl
from jax.experimental.pallas import tpu as pltpu
from jax.experimental.pallas import tpu_sc as plsc

_N = 10000
_E = 320000
_D = 64
_NC, _NS = 2, 16
_NW = _NC * _NS
_CH = 128
_NCH = 80
_EP = _NW * _NCH * _CH
_PAD = _EP - _E
_NP = 10240
_RPT = _NP // _NS
_ZR = 128
_RB = 1000

_mesh = plsc.VectorSubcoreMesh(
    core_axis_name="c", subcore_axis_name="s",
    num_cores=_NC, num_subcores=_NS)


def _deg_body(dst_hbm, out_hbm, dstv, onesv, zbuf, acc):
    c = lax.axis_index("c")
    s = lax.axis_index("s")
    wid = s * _NC + c

    @pl.loop(0, _CH)
    def _fill(i):
        onesv[i, :] = jnp.full((16,), 1.0, dtype=jnp.float32)

    @pl.loop(0, _RPT)
    def _zero(i):
        zbuf[i, :] = jnp.zeros((16,), dtype=jnp.float32)

    pltpu.sync_copy(dst_hbm.at[wid], dstv)
    pltpu.sync_copy(zbuf, acc.at[pl.ds(s * _RPT, _RPT)])
    plsc.subcore_barrier()

    @pl.loop(0, _NCH)
    def _scat(i):
        pltpu.sync_copy(onesv, acc.at[dstv.at[i]], add=True)

    plsc.subcore_barrier()
    pltpu.sync_copy(acc.at[pl.ds(s * _RPT, _RPT)],
                    out_hbm.at[c].at[pl.ds(s * _RPT, _RPT)])


def _deg_partials(dst):
    return pl.kernel(
        _deg_body,
        out_type=jax.ShapeDtypeStruct((_NC, _NP, 16), jnp.float32),
        mesh=_mesh,
        compiler_params=pltpu.CompilerParams(use_tc_tiling_on_sc=False),
        scratch_types=[
            pltpu.VMEM((_NCH, _CH), jnp.int32),
            pltpu.VMEM((_CH, 16), jnp.float32),
            pltpu.VMEM((_RPT, 16), jnp.float32),
            pltpu.VMEM_SHARED((_NP, 16), jnp.float32),
        ],
    )(dst)


def _agg_body(g_hbm, src_hbm, dst_hbm, out_hbm,
              srcv, dstv, bufs, zbuf, acc, gsems, ssems):
    c = lax.axis_index("c")
    s = lax.axis_index("s")
    wid = s * _NC + c

    @pl.loop(0, _ZR)
    def _zero(i):
        @pl.loop(0, _D // 16)
        def _zero_in(j):
            zbuf[i, pl.ds(j * 16, 16)] = jnp.zeros((16,), dtype=jnp.float32)

    @pl.loop(0, _RPT // _ZR)
    def _zacc(j):
        pltpu.sync_copy(zbuf, acc.at[pl.ds(s * _RPT + j * _ZR, _ZR)])

    pltpu.sync_copy(src_hbm.at[wid], srcv)
    pltpu.sync_copy(dst_hbm.at[wid], dstv)
    plsc.subcore_barrier()

    @pl.loop(0, _NCH, step=4)
    def _pipe(i):
        gs = [pltpu.async_copy(g_hbm.at[srcv.at[i + k]], bufs[k], gsems[k])
              for k in range(4)]
        ss = []
        for k in range(4):
            gs[k].wait()
            ss.append(pltpu.async_copy(bufs[k], acc.at[dstv.at[i + k]],
                                       ssems[k], add=True))
        for k in range(4):
            ss[k].wait()

    plsc.subcore_barrier()
    pltpu.sync_copy(acc.at[pl.ds(s * _RPT, _RPT)],
                    out_hbm.at[c].at[pl.ds(s * _RPT, _RPT)])


def _aggregate(g, src, dst):
    return pl.kernel(
        _agg_body2,
        out_type=jax.ShapeDtypeStruct((_NC, _NP, _D), jnp.float32),
        mesh=_mesh,
        compiler_params=pltpu.CompilerParams(use_tc_tiling_on_sc=False),
        scratch_types=[
            pltpu.VMEM((_NCH, _CH), jnp.int32),
            pltpu.VMEM((_NCH, _CH), jnp.int32),
            pltpu.VMEM((_CH, _D), jnp.float32),
            pltpu.VMEM((_CH, _D), jnp.float32),
            pltpu.VMEM((_CH, _D), jnp.float32),
            pltpu.VMEM((_CH, _D), jnp.float32),
            pltpu.VMEM((_ZR, _D), jnp.float32),
            pltpu.VMEM_SHARED((_NP, _D), jnp.float32),
        ] + [pltpu.SemaphoreType.DMA] * 8,
    )(g, src, dst)


def _agg_body2(g_hbm, src_hbm, dst_hbm, out_hbm,
               srcv, dstv, b0, b1, b2, b3, zbuf, acc,
               g0, g1, g2, g3, s0, s1, s2, s3):
    _agg_body(g_hbm, src_hbm, dst_hbm, out_hbm,
              srcv, dstv, [b0, b1, b2, b3], zbuf, acc,
              [g0, g1, g2, g3], [s0, s1, s2, s3])


def _matmul1(x, W1):
    def body(x_ref, w_ref, o_ref):
        o_ref[...] = jnp.dot(x_ref[...], w_ref[...],
                             preferred_element_type=jnp.float32)

    return pl.pallas_call(
        body,
        grid=(_N // _RB,),
        in_specs=[pl.BlockSpec((_RB, 128), lambda i: (i, 0)),
                  pl.BlockSpec((128, _D), lambda i: (0, 0))],
        out_specs=pl.BlockSpec((_RB, _D), lambda i: (i, 0)),
        out_shape=jax.ShapeDtypeStruct((_N, _D), jnp.float32),
    )(x, W1)


def _combine1(degp, h1, b1):
    def body(p_ref, h_ref, b_ref, g_ref, st_ref, dis_ref):
        deg = 1.0 + p_ref[0] + p_ref[1]
        dcol = lax.rsqrt(deg[:, 0:1])
        disb = jnp.broadcast_to(dcol, (_RB, _D))
        h = h_ref[...]
        g_ref[...] = h * disb
        st_ref[...] = h * (dcol * dcol) + b_ref[...]
        dis_ref[...] = disb

    return pl.pallas_call(
        body,
        grid=(_N // _RB,),
        in_specs=[pl.BlockSpec((_NC, _RB, 16), lambda i: (0, i, 0)),
                  pl.BlockSpec((_RB, _D), lambda i: (i, 0)),
                  pl.BlockSpec((1, _D), lambda i: (0, 0))],
        out_specs=[pl.BlockSpec((_RB, _D), lambda i: (i, 0))] * 3,
        out_shape=[jax.ShapeDtypeStruct((_N, _D), jnp.float32)] * 3,
    )(degp, h1, b1)


def _combine2(s1, disb, st1):
    def body(p_ref, d_ref, st_ref, g_ref, st2_ref):
        dis = d_ref[...]
        h = jnp.maximum(dis * (p_ref[0] + p_ref[1]) + st_ref[...], 0.0)
        g_ref[...] = h * dis
        st2_ref[...] = h * dis * dis

    return pl.pallas_call(
        body,
        grid=(_N // _RB,),
        in_specs=[pl.BlockSpec((_NC, _RB, _D), lambda i: (0, i, 0)),
                  pl.BlockSpec((_RB, _D), lambda i: (i, 0)),
                  pl.BlockSpec((_RB, _D), lambda i: (i, 0))],
        out_specs=[pl.BlockSpec((_RB, _D), lambda i: (i, 0))] * 2,
        out_shape=[jax.ShapeDtypeStruct((_N, _D), jnp.float32)] * 2,
    )(s1, disb, st1)


def _final(s2, disb, st2, W2, b2):
    def body(p_ref, d_ref, st_ref, w_ref, b_ref, o_ref):
        a2 = d_ref[...] * (p_ref[0] + p_ref[1]) + st_ref[...]
        o_ref[...] = jnp.dot(a2, w_ref[...],
                             preferred_element_type=jnp.float32) + b_ref[...]

    return pl.pallas_call(
        body,
        grid=(_N // _RB,),
        in_specs=[pl.BlockSpec((_NC, _RB, _D), lambda i: (0, i, 0)),
                  pl.BlockSpec((_RB, _D), lambda i: (i, 0)),
                  pl.BlockSpec((_RB, _D), lambda i: (i, 0)),
                  pl.BlockSpec((_D, 128), lambda i: (0, 0)),
                  pl.BlockSpec((1, 128), lambda i: (0, 0))],
        out_specs=pl.BlockSpec((_RB, 128), lambda i: (i, 0)),
        out_shape=jax.ShapeDtypeStruct((_N, 128), jnp.float32),
    )(s2, disb, st2, W2, b2)


def kernel(x, edge_index, W1, b1, W2, b2):
    ei = edge_index.astype(jnp.int32)
    pad_src = jnp.arange(_PAD, dtype=jnp.int32) % _N
    pad_dst = _N + jnp.arange(_PAD, dtype=jnp.int32) % (_NP - _N)
    src = jnp.concatenate([ei[0], pad_src]).reshape(_NW, _NCH, _CH)
    dst = jnp.concatenate([ei[1], pad_dst]).reshape(_NW, _NCH, _CH)

    degp = _deg_partials(dst)
    h1 = _matmul1(x, W1)

    g1, st1, disb = _combine1(degp, h1, b1.reshape(1, _D))
    s1 = _aggregate(g1, src, dst)
    g2, st2 = _combine2(s1, disb, st1)
    s2 = _aggregate(g2, src, dst)
    return _final(s2, disb, st2, W2, b2.reshape(1, 128))
